# baseline (speedup 1.0000x reference)
import jax
import jax.numpy as jnp
from jax import lax
from jax.experimental import pallas as pl
from jax.experimental.pallas import tpu as pltpu
from jax.experimental.pallas import tpu_sc as plsc

B = 16384
L = 200
LS = 20
DD, DP, DS = 32, 16, 16
FEAT = (DD + DP + DS) * 2
H1, H2 = 128, 64

NC, NS = 2, 16
NW = NC * NS
RPW = B // NW
SB = 64
NSB = RPW // SB
NB = 8
K = 7


OFF_CD = 0
OFF_CP = B * L
OFF_CS = 2 * B * L
OFF_PD = 2 * B * L + B * LS
OFF_PP = OFF_PD + B * L
OFF_PS = OFF_PP + B * L


def _sc_featurize_body(idx_all, dis_hbm, phe_hbm, sub_hbm, out_hbm,
                       dis_t, phe_t, sub_t, idx_v, rows32, rows16, stage, *sems):
    wid = lax.axis_index("s") * NC + lax.axis_index("c")
    wrow = wid * RPW
    sid = lax.axis_index("s")

    nd = dis_hbm.shape[0] // NS
    np_ = phe_hbm.shape[0] // NS
    pltpu.sync_copy(dis_hbm.at[pl.ds(sid * nd, nd), :], dis_t.at[pl.ds(sid * nd, nd), :])
    pltpu.sync_copy(phe_hbm.at[pl.ds(sid * np_, np_), :], phe_t.at[pl.ds(sid * np_, np_), :])

    @pl.when(sid == 0)
    def _():
        rem_d = dis_hbm.shape[0] - nd * NS
        rem_p = phe_hbm.shape[0] - np_ * NS
        pltpu.sync_copy(dis_hbm.at[pl.ds(nd * NS, rem_d), :], dis_t.at[pl.ds(nd * NS, rem_d), :])
        pltpu.sync_copy(phe_hbm.at[pl.ds(np_ * NS, rem_p), :], phe_t.at[pl.ds(np_ * NS, rem_p), :])
        pltpu.sync_copy(sub_hbm, sub_t)

    plsc.subcore_barrier()

    def long_pass(pass_off, tab_sp, d, col):
        wbase = pass_off + wid * (RPW * L)
        rows = rows32 if d == 32 else rows16
        inv = 1.0 / L
        NCH = SB * 2

        def fire(ch_q, ch_i, buf, sem):
            r = ch_q * 4 + ch_i // 2
            if ch_i % 2 == 0:
                o = pl.multiple_of(r * L, 8)
                pltpu.async_copy(tab_sp.at[idx_v.at[pl.ds(o, 128)]],
                                 rows.at[buf, pl.ds(0, 128), :], sem)
            else:
                o = pl.multiple_of(r * L + 128, 8)
                pltpu.async_copy(tab_sp.at[idx_v.at[pl.ds(o, 72)]],
                                 rows.at[buf, pl.ds(0, 72), :], sem)

        def drain(part, buf, sem):
            n = 128 if part == 0 else 72
            pltpu.make_async_copy(tab_sp.at[idx_v.at[pl.ds(0, n)]],
                                  rows.at[buf, pl.ds(0, n), :], sem).wait()

        def reduce_chunk(buf, nrows, acc):
            if d == 32:
                def red(j, a):
                    a0, a1, b0, b1 = a
                    base = j * 8
                    for t in range(0, 8, 2):
                        x0 = rows[buf, base + t, :]
                        x1 = rows[buf, base + t + 1, :]
                        a0 = a0 + lax.bitcast_convert_type(x0 << 16, jnp.float32)
                        a1 = a1 + lax.bitcast_convert_type(x0 & jnp.int32(-65536), jnp.float32)
                        b0 = b0 + lax.bitcast_convert_type(x1 << 16, jnp.float32)
                        b1 = b1 + lax.bitcast_convert_type(x1 & jnp.int32(-65536), jnp.float32)
                    return a0, a1, b0, b1
            else:
                def red(j, a):
                    a0, b0 = a
                    base = j * 8
                    for t in range(0, 8, 2):
                        r0 = base + t
                        a0 = a0 + rows[buf, r0, pl.ds(0, 16)]
                        b0 = b0 + rows[buf, r0 + 1, pl.ds(0, 16)]
                    return a0, b0
            return lax.fori_loop(0, nrows // 8, red, acc)

        def store_row(row, acc):
            if d == 32:
                a0, a1, b0, b1 = acc
                stage[row, pl.ds(0, 16)] = (a0 + b0) * inv
                stage[row, pl.ds(16, 16)] = (a1 + b1) * inv
            else:
                a0, b0 = acc
                stage[row, pl.ds(0, 16)] = (a0 + b0) * inv

        zacc = (jnp.zeros((16,), jnp.float32),) * (4 if d == 32 else 2)

        def sblock_body(sb, _):
            boff = pl.multiple_of(wbase + sb * (SB * L), 8)
            pltpu.sync_copy(idx_all.at[pl.ds(boff, SB * L)],
                            idx_v.at[pl.ds(0, SB * L)])
            for p in range(K):
                fire(0, p, p % NB, sems[p % NB])

            def oct_body(q, _):
                acc = zacc
                for i in range(NB):
                    fi = i + K
                    fbuf = (i + K) % NB

                    @pl.when(q * NB + fi < NCH)
                    def _():
                        fire(q, fi, fbuf, sems[fbuf])

                    drain(i % 2, i, sems[i])
                    acc = reduce_chunk(i, 128 if i % 2 == 0 else 72, acc)
                    if i % 2 == 1:
                        store_row(sb * SB + q * 4 + i // 2, acc)
                        acc = zacc
                return 0

            lax.fori_loop(0, NCH // NB, oct_body, 0)
            return 0

        lax.fori_loop(0, NSB, sblock_body, 0)
        src = stage if d == 32 else stage.at[:, pl.ds(0, 16)]
        pltpu.sync_copy(src, out_hbm.at[pl.ds(wrow, RPW), pl.ds(col, d)])

    def fire_s(buf, off, sem):
        o = pl.multiple_of(off, 8)
        pltpu.async_copy(sub_t.at[idx_v.at[pl.ds(o, 80)]],
                         rows16.at[buf, pl.ds(0, 80), :], sem)

    def drain_s(buf, sem):
        pltpu.make_async_copy(sub_t.at[idx_v.at[pl.ds(0, 80)]],
                              rows16.at[buf, pl.ds(0, 80), :], sem).wait()

    def sub_pass(pass_off, col):
        wbase = pass_off + wid * (RPW * LS)
        inv = 1.0 / LS
        nchunks = RPW // 4
        pltpu.sync_copy(idx_all.at[pl.ds(pl.multiple_of(wbase, 8), RPW * LS)],
                        idx_v.at[pl.ds(0, RPW * LS)])
        for p in range(4):
            fire_s(p % NB, p * 80, sems[p % NB])

        def oct_body(q, _):
            for i in range(NB):
                c = q * NB + i
                fc = c + 4
                fbuf = (i + 4) % NB

                @pl.when(fc < nchunks)
                def _():
                    fire_s(fbuf, fc * 80, sems[fbuf])

                drain_s(i, sems[i])
                for seg in range(4):
                    acc = jnp.zeros((16,), jnp.float32)
                    for j in range(LS):
                        acc = acc + rows16[i, seg * LS + j, pl.ds(0, 16)]
                    stage[c * 4 + seg, pl.ds(0, 16)] = acc * inv
            return 0

        lax.fori_loop(0, nchunks // NB, oct_body, 0)
        pltpu.sync_copy(stage.at[:, pl.ds(0, 16)],
                        out_hbm.at[pl.ds(wrow, RPW), pl.ds(col, 16)])

    long_pass(OFF_CD, dis_t, 32, 0)
    long_pass(OFF_CP, phe_t, 16, 32)
    sub_pass(OFF_CS, 48)
    long_pass(OFF_PD, dis_t, 32, 64)
    long_pass(OFF_PP, phe_t, 16, 96)
    sub_pass(OFF_PS, 112)


def _sc_featurize(idx_all, dis_t, phe_t, sub_t):
    mesh = plsc.VectorSubcoreMesh(core_axis_name="c", subcore_axis_name="s")
    f = pl.kernel(
        _sc_featurize_body,
        out_type=jax.ShapeDtypeStruct((B, FEAT), jnp.float32),
        mesh=mesh,
        compiler_params=pltpu.CompilerParams(use_tc_tiling_on_sc=False),
        scratch_types=[
            pltpu.VMEM_SHARED((13752, 16), jnp.int32),
            pltpu.VMEM_SHARED((17393, 16), jnp.float32),
            pltpu.VMEM_SHARED((30, 16), jnp.float32),
            pltpu.VMEM((SB * L,), jnp.int32),
            pltpu.VMEM((NB, 128, 16), jnp.int32),
            pltpu.VMEM((NB, 128, 16), jnp.float32),
            pltpu.VMEM((RPW, 32), jnp.float32),
        ] + [pltpu.SemaphoreType.DMA] * NB,
    )
    return f(idx_all, dis_t, phe_t, sub_t)


def _mlp_body(x_ref, w1_ref, b1_ref, w2_ref, b2_ref, w3_ref, b3_ref, o_ref):
    x = x_ref[...]
    h = jnp.dot(x, w1_ref[...], preferred_element_type=jnp.float32)
    h = h + b1_ref[...]
    h = jnp.where(h >= 0, h, 0.01 * h)
    h = jnp.dot(h, w2_ref[...], preferred_element_type=jnp.float32)
    h = h + b2_ref[...]
    h = jnp.where(h >= 0, h, 0.01 * h)
    o = jnp.dot(h, w3_ref[...], preferred_element_type=jnp.float32)
    o_ref[...] = o + b3_ref[...]


def _mlp(x, W1, b1, W2, b2, W3, b3):
    BB = 1024
    return pl.pallas_call(
        _mlp_body,
        grid=(B // BB,),
        in_specs=[
            pl.BlockSpec((BB, FEAT), lambda i: (i, 0)),
            pl.BlockSpec((FEAT, H1), lambda i: (0, 0)),
            pl.BlockSpec((1, H1), lambda i: (0, 0)),
            pl.BlockSpec((H1, H2), lambda i: (0, 0)),
            pl.BlockSpec((1, H2), lambda i: (0, 0)),
            pl.BlockSpec((H2, 1), lambda i: (0, 0)),
            pl.BlockSpec((1, 1), lambda i: (0, 0)),
        ],
        out_specs=pl.BlockSpec((BB, 1), lambda i: (i, 0)),
        out_shape=jax.ShapeDtypeStruct((B, 1), jnp.float32),
    )(x, W1, b1.reshape(1, H1), W2, b2.reshape(1, H2), W3, b3.reshape(1, 1))


def kernel(compound_diseases, compound_phenotypes, compound_subcellular_locations,
           protein_diseases, protein_phenotypes, protein_subcellular_locations,
           disease_table, phenotype_table, sub_table, W1, b1, W2, b2, W3, b3):
    idx_all = jnp.concatenate([
        compound_diseases.reshape(-1).astype(jnp.int32),
        compound_phenotypes.reshape(-1).astype(jnp.int32),
        compound_subcellular_locations.reshape(-1).astype(jnp.int32),
        protein_diseases.reshape(-1).astype(jnp.int32),
        protein_phenotypes.reshape(-1).astype(jnp.int32),
        protein_subcellular_locations.reshape(-1).astype(jnp.int32),
    ])
    perm = jnp.arange(32).reshape(2, 16).T.reshape(-1)
    dis_packed = jax.lax.bitcast_convert_type(
        disease_table.astype(jnp.bfloat16)[:, perm].reshape(-1, 16, 2),
        jnp.int32)
    x = _sc_featurize(idx_all, dis_packed, phenotype_table, sub_table)
    return _mlp(x, W1, b1, W2, b2, W3, b3)


# speedup vs baseline: 1.0617x; 1.0617x over previous
import jax
import jax.numpy as jnp
from jax import lax
from jax.experimental import pallas as pl
from jax.experimental.pallas import tpu as pltpu
from jax.experimental.pallas import tpu_sc as plsc

B = 16384
L = 200
LS = 20
DD, DP, DS = 32, 16, 16
FEAT = (DD + DP + DS) * 2
H1, H2 = 128, 64

NC, NS = 2, 16
NW = NC * NS
RPW = B // NW
SB = 64
NSB = RPW // SB
NB = 8
K = 7


def _sc_featurize_body(cd, cp, cs, pd, pp, ps, dis_hbm, phe_hbm, sub_hbm, out_hbm,
                       dis_t, phe_t, sub_t, idx_v, idx_s, rows32, rows16, stage, *sems):
    wid = lax.axis_index("s") * NC + lax.axis_index("c")
    wrow = wid * RPW
    sid = lax.axis_index("s")

    nd = dis_hbm.shape[0] // NS
    np_ = phe_hbm.shape[0] // NS
    pltpu.sync_copy(dis_hbm.at[pl.ds(sid * nd, nd), :], dis_t.at[pl.ds(sid * nd, nd), :])
    pltpu.sync_copy(phe_hbm.at[pl.ds(sid * np_, np_), :], phe_t.at[pl.ds(sid * np_, np_), :])

    @pl.when(sid == 0)
    def _():
        rem_d = dis_hbm.shape[0] - nd * NS
        rem_p = phe_hbm.shape[0] - np_ * NS
        pltpu.sync_copy(dis_hbm.at[pl.ds(nd * NS, rem_d), :], dis_t.at[pl.ds(nd * NS, rem_d), :])
        pltpu.sync_copy(phe_hbm.at[pl.ds(np_ * NS, rem_p), :], phe_t.at[pl.ds(np_ * NS, rem_p), :])
        pltpu.sync_copy(sub_hbm, sub_t)

    plsc.subcore_barrier()

    def long_pass(idx_hbm, tab_sp, d, col):
        rows = rows32 if d == 32 else rows16
        inv = 1.0 / L
        NCH = SB * 2

        def fire(ch_q, ch_i, buf, sem):
            r = ch_q * 4 + ch_i // 2
            if ch_i % 2 == 0:
                pltpu.async_copy(tab_sp.at[idx_v.at[r, pl.ds(0, 128)]],
                                 rows.at[buf, pl.ds(0, 128), :], sem)
            else:
                pltpu.async_copy(tab_sp.at[idx_v.at[r, pl.ds(128, 72)]],
                                 rows.at[buf, pl.ds(0, 72), :], sem)

        def drain(part, buf, sem):
            n = 128 if part == 0 else 72
            pltpu.make_async_copy(tab_sp.at[idx_v.at[0, pl.ds(0, n)]],
                                  rows.at[buf, pl.ds(0, n), :], sem).wait()

        def reduce_chunk(buf, nrows, acc):
            if d == 32:
                def red(j, a):
                    a0, a1, b0, b1 = a
                    base = j * 8
                    for t in range(0, 8, 2):
                        x0 = rows[buf, base + t, :]
                        x1 = rows[buf, base + t + 1, :]
                        a0 = a0 + lax.bitcast_convert_type(x0 << 16, jnp.float32)
                        a1 = a1 + lax.bitcast_convert_type(x0 & jnp.int32(-65536), jnp.float32)
                        b0 = b0 + lax.bitcast_convert_type(x1 << 16, jnp.float32)
                        b1 = b1 + lax.bitcast_convert_type(x1 & jnp.int32(-65536), jnp.float32)
                    return a0, a1, b0, b1
            else:
                def red(j, a):
                    a0, b0 = a
                    base = j * 8
                    for t in range(0, 8, 2):
                        r0 = base + t
                        a0 = a0 + rows[buf, r0, pl.ds(0, 16)]
                        b0 = b0 + rows[buf, r0 + 1, pl.ds(0, 16)]
                    return a0, b0
            return lax.fori_loop(0, nrows // 8, red, acc)

        def store_row(row, acc):
            if d == 32:
                a0, a1, b0, b1 = acc
                stage[row, pl.ds(0, 16)] = (a0 + b0) * inv
                stage[row, pl.ds(16, 16)] = (a1 + b1) * inv
            else:
                a0, b0 = acc
                stage[row, pl.ds(0, 16)] = (a0 + b0) * inv

        zacc = (jnp.zeros((16,), jnp.float32),) * (4 if d == 32 else 2)

        def sblock_body(sb, _):
            row0 = wrow + sb * SB
            pltpu.sync_copy(idx_hbm.at[pl.ds(row0, SB), :], idx_v)
            for p in range(K):
                fire(0, p, p % NB, sems[p % NB])

            def oct_body(q, _):
                acc = zacc
                for i in range(NB):
                    fi = i + K
                    fbuf = (i + K) % NB

                    @pl.when(q * NB + fi < NCH)
                    def _():
                        fire(q, fi, fbuf, sems[fbuf])

                    drain(i % 2, i, sems[i])
                    acc = reduce_chunk(i, 128 if i % 2 == 0 else 72, acc)
                    if i % 2 == 1:
                        store_row(sb * SB + q * 4 + i // 2, acc)
                        acc = zacc
                return 0

            lax.fori_loop(0, NCH // NB, oct_body, 0)
            return 0

        lax.fori_loop(0, NSB, sblock_body, 0)
        src = stage if d == 32 else stage.at[:, pl.ds(0, 16)]
        pltpu.sync_copy(src, out_hbm.at[pl.ds(wrow, RPW), pl.ds(col, d)])

    def fire_s(buf, off, sem):
        o = pl.multiple_of(off, 8)
        pltpu.async_copy(sub_t.at[idx_s.at[pl.ds(o, 80)]],
                         rows16.at[buf, pl.ds(0, 80), :], sem)

    def drain_s(buf, sem):
        pltpu.make_async_copy(sub_t.at[idx_s.at[pl.ds(0, 80)]],
                              rows16.at[buf, pl.ds(0, 80), :], sem).wait()

    def sub_pass(idx_hbm, col):
        wbase = wid * (RPW * LS)
        inv = 1.0 / LS
        nchunks = RPW // 4
        pltpu.sync_copy(idx_hbm.at[pl.ds(pl.multiple_of(wbase, 8), RPW * LS)],
                        idx_s)
        for p in range(4):
            fire_s(p % NB, p * 80, sems[p % NB])

        def oct_body(q, _):
            for i in range(NB):
                c = q * NB + i
                fc = c + 4
                fbuf = (i + 4) % NB

                @pl.when(fc < nchunks)
                def _():
                    fire_s(fbuf, fc * 80, sems[fbuf])

                drain_s(i, sems[i])
                for seg in range(4):
                    acc = jnp.zeros((16,), jnp.float32)
                    for j in range(LS):
                        acc = acc + rows16[i, seg * LS + j, pl.ds(0, 16)]
                    stage[c * 4 + seg, pl.ds(0, 16)] = acc * inv
            return 0

        lax.fori_loop(0, nchunks // NB, oct_body, 0)
        pltpu.sync_copy(stage.at[:, pl.ds(0, 16)],
                        out_hbm.at[pl.ds(wrow, RPW), pl.ds(col, 16)])

    long_pass(cd, dis_t, 32, 0)
    long_pass(cp, phe_t, 16, 32)
    sub_pass(cs, 48)
    long_pass(pd, dis_t, 32, 64)
    long_pass(pp, phe_t, 16, 96)
    sub_pass(ps, 112)


def _sc_featurize(cd, cp, cs, pd, pp, ps, dis_t, phe_t, sub_t):
    mesh = plsc.VectorSubcoreMesh(core_axis_name="c", subcore_axis_name="s")
    f = pl.kernel(
        _sc_featurize_body,
        out_type=jax.ShapeDtypeStruct((B, FEAT), jnp.float32),
        mesh=mesh,
        compiler_params=pltpu.CompilerParams(use_tc_tiling_on_sc=False),
        scratch_types=[
            pltpu.VMEM_SHARED((13752, 16), jnp.int32),
            pltpu.VMEM_SHARED((17393, 16), jnp.float32),
            pltpu.VMEM_SHARED((30, 16), jnp.float32),
            pltpu.VMEM((SB, L), jnp.int32),
            pltpu.VMEM((RPW * LS,), jnp.int32),
            pltpu.VMEM((NB, 128, 16), jnp.int32),
            pltpu.VMEM((NB, 128, 16), jnp.float32),
            pltpu.VMEM((RPW, 32), jnp.float32),
        ] + [pltpu.SemaphoreType.DMA] * NB,
    )
    return f(cd, cp, cs, pd, pp, ps, dis_t, phe_t, sub_t)


def _mlp_body(x_ref, w1_ref, b1_ref, w2_ref, b2_ref, w3_ref, b3_ref, o_ref):
    x = x_ref[...]
    h = jnp.dot(x, w1_ref[...], preferred_element_type=jnp.float32)
    h = h + b1_ref[...]
    h = jnp.where(h >= 0, h, 0.01 * h)
    h = jnp.dot(h, w2_ref[...], preferred_element_type=jnp.float32)
    h = h + b2_ref[...]
    h = jnp.where(h >= 0, h, 0.01 * h)
    o = jnp.dot(h, w3_ref[...], preferred_element_type=jnp.float32)
    o_ref[...] = o + b3_ref[...]


def _mlp(x, W1, b1, W2, b2, W3, b3):
    BB = 1024
    return pl.pallas_call(
        _mlp_body,
        grid=(B // BB,),
        in_specs=[
            pl.BlockSpec((BB, FEAT), lambda i: (i, 0)),
            pl.BlockSpec((FEAT, H1), lambda i: (0, 0)),
            pl.BlockSpec((1, H1), lambda i: (0, 0)),
            pl.BlockSpec((H1, H2), lambda i: (0, 0)),
            pl.BlockSpec((1, H2), lambda i: (0, 0)),
            pl.BlockSpec((H2, 1), lambda i: (0, 0)),
            pl.BlockSpec((1, 1), lambda i: (0, 0)),
        ],
        out_specs=pl.BlockSpec((BB, 1), lambda i: (i, 0)),
        out_shape=jax.ShapeDtypeStruct((B, 1), jnp.float32),
    )(x, W1, b1.reshape(1, H1), W2, b2.reshape(1, H2), W3, b3.reshape(1, 1))


def kernel(compound_diseases, compound_phenotypes, compound_subcellular_locations,
           protein_diseases, protein_phenotypes, protein_subcellular_locations,
           disease_table, phenotype_table, sub_table, W1, b1, W2, b2, W3, b3):
    cd = compound_diseases.astype(jnp.int32)
    cp = compound_phenotypes.astype(jnp.int32)
    cs = compound_subcellular_locations.reshape(-1).astype(jnp.int32)
    pd = protein_diseases.astype(jnp.int32)
    pp = protein_phenotypes.astype(jnp.int32)
    ps = protein_subcellular_locations.reshape(-1).astype(jnp.int32)
    perm = jnp.arange(32).reshape(2, 16).T.reshape(-1)
    dis_packed = jax.lax.bitcast_convert_type(
        disease_table.astype(jnp.bfloat16)[:, perm].reshape(-1, 16, 2),
        jnp.int32)
    x = _sc_featurize(cd, cp, cs, pd, pp, ps, dis_packed, phenotype_table, sub_table)
    return _mlp(x, W1, b1, W2, b2, W3, b3)


# speedup vs baseline: 1.1254x; 1.0600x over previous
import jax
import jax.numpy as jnp
from jax import lax
from jax.experimental import pallas as pl
from jax.experimental.pallas import tpu as pltpu
from jax.experimental.pallas import tpu_sc as plsc

B = 16384
L = 200
LS = 20
DD, DP, DS = 32, 16, 16
FEAT = (DD + DP + DS) * 2
H1, H2 = 128, 64

NC, NS = 2, 16
NW = NC * NS
RPW = B // NW
SB = 64
NSB = RPW // SB
NB = 8
K = 7


def _sc_featurize_body(idx_d, idx_p, idx_sub, dis_hbm, phe_hbm, sub_hbm, out_hbm,
                       dis_t, phe_t, sub_t, idx_v, idx_s, rows32, rows16, stage, *sems):
    wid = lax.axis_index("s") * NC + lax.axis_index("c")
    wrow = wid * RPW
    sid = lax.axis_index("s")

    nd = dis_hbm.shape[0] // NS
    np_ = phe_hbm.shape[0] // NS
    pltpu.sync_copy(dis_hbm.at[pl.ds(sid * nd, nd), :], dis_t.at[pl.ds(sid * nd, nd), :])
    pltpu.sync_copy(phe_hbm.at[pl.ds(sid * np_, np_), :], phe_t.at[pl.ds(sid * np_, np_), :])

    @pl.when(sid == 0)
    def _():
        rem_d = dis_hbm.shape[0] - nd * NS
        rem_p = phe_hbm.shape[0] - np_ * NS
        pltpu.sync_copy(dis_hbm.at[pl.ds(nd * NS, rem_d), :], dis_t.at[pl.ds(nd * NS, rem_d), :])
        pltpu.sync_copy(phe_hbm.at[pl.ds(np_ * NS, rem_p), :], phe_t.at[pl.ds(np_ * NS, rem_p), :])
        pltpu.sync_copy(sub_hbm, sub_t)

    plsc.subcore_barrier()

    def long_pass(idx_hbm, tab_sp, d, col):
        rows = rows32 if d == 32 else rows16
        inv = 1.0 / L
        NCH = SB * 2

        def fire(ch_q, ch_i, buf, sem):
            r = ch_q * 4 + ch_i // 2
            if ch_i % 2 == 0:
                pltpu.async_copy(tab_sp.at[idx_v.at[r, pl.ds(0, 128)]],
                                 rows.at[buf, pl.ds(0, 128), :], sem)
            else:
                pltpu.async_copy(tab_sp.at[idx_v.at[r, pl.ds(128, 72)]],
                                 rows.at[buf, pl.ds(0, 72), :], sem)

        def drain(part, buf, sem):
            n = 128 if part == 0 else 72
            pltpu.make_async_copy(tab_sp.at[idx_v.at[0, pl.ds(0, n)]],
                                  rows.at[buf, pl.ds(0, n), :], sem).wait()

        def reduce_chunk(buf, nrows, acc):
            if d == 32:
                def red(j, a):
                    a0, a1, b0, b1 = a
                    base = j * 8
                    for t in range(0, 8, 2):
                        x0 = rows[buf, base + t, :]
                        x1 = rows[buf, base + t + 1, :]
                        a0 = a0 + lax.bitcast_convert_type(x0 << 16, jnp.float32)
                        a1 = a1 + lax.bitcast_convert_type(x0 & jnp.int32(-65536), jnp.float32)
                        b0 = b0 + lax.bitcast_convert_type(x1 << 16, jnp.float32)
                        b1 = b1 + lax.bitcast_convert_type(x1 & jnp.int32(-65536), jnp.float32)
                    return a0, a1, b0, b1
            else:
                def red(j, a):
                    a0, b0 = a
                    base = j * 8
                    for t in range(0, 8, 2):
                        r0 = base + t
                        a0 = a0 + rows[buf, r0, pl.ds(0, 16)]
                        b0 = b0 + rows[buf, r0 + 1, pl.ds(0, 16)]
                    return a0, b0
            return lax.fori_loop(0, nrows // 8, red, acc)

        def store_row(row, acc):
            if d == 32:
                a0, a1, b0, b1 = acc
                stage[row, pl.ds(0, 16)] = (a0 + b0) * inv
                stage[row, pl.ds(16, 16)] = (a1 + b1) * inv
            else:
                a0, b0 = acc
                stage[row, pl.ds(0, 16)] = (a0 + b0) * inv

        zacc = (jnp.zeros((16,), jnp.float32),) * (4 if d == 32 else 2)

        def sblock_body(sb, _):
            row0 = wrow + sb * SB
            pltpu.sync_copy(idx_hbm.at[pl.ds(row0, SB), :], idx_v)
            for p in range(K):
                fire(0, p, p % NB, sems[p % NB])

            def oct_body(q, _):
                acc = zacc
                for i in range(NB):
                    fi = i + K
                    fbuf = (i + K) % NB

                    @pl.when(q * NB + fi < NCH)
                    def _():
                        fire(q, fi, fbuf, sems[fbuf])

                    drain(i % 2, i, sems[i])
                    acc = reduce_chunk(i, 128 if i % 2 == 0 else 72, acc)
                    if i % 2 == 1:
                        store_row(sb * SB + q * 4 + i // 2, acc)
                        acc = zacc
                return 0

            lax.fori_loop(0, NCH // NB, oct_body, 0)
            return 0

        lax.fori_loop(0, NSB, sblock_body, 0)
        src = stage if d == 32 else stage.at[:, pl.ds(0, 16)]
        pltpu.sync_copy(src, out_hbm.at[pl.ds(wrow, RPW), pl.ds(col, d)])

    def fire_s(buf, off, sem):
        o = pl.multiple_of(off, 8)
        pltpu.async_copy(sub_t.at[idx_s.at[pl.ds(o, 80)]],
                         rows16.at[buf, pl.ds(0, 80), :], sem)

    def drain_s(buf, sem):
        pltpu.make_async_copy(sub_t.at[idx_s.at[pl.ds(0, 80)]],
                              rows16.at[buf, pl.ds(0, 80), :], sem).wait()

    def sub_pass(idx_hbm, col):
        wbase = wid * (RPW * LS)
        inv = 1.0 / LS
        nchunks = RPW // 4
        pltpu.sync_copy(idx_hbm.at[pl.ds(pl.multiple_of(wbase, 8), RPW * LS)],
                        idx_s)
        for p in range(4):
            fire_s(p % NB, p * 80, sems[p % NB])

        def oct_body(q, _):
            for i in range(NB):
                c = q * NB + i
                fc = c + 4
                fbuf = (i + 4) % NB

                @pl.when(fc < nchunks)
                def _():
                    fire_s(fbuf, fc * 80, sems[fbuf])

                drain_s(i, sems[i])
                for seg in range(4):
                    acc = jnp.zeros((16,), jnp.float32)
                    for j in range(LS):
                        acc = acc + rows16[i, seg * LS + j, pl.ds(0, 16)]
                    stage[c * 4 + seg, pl.ds(0, 16)] = acc * inv
            return 0

        lax.fori_loop(0, nchunks // NB, oct_body, 0)
        pltpu.sync_copy(stage.at[:, pl.ds(0, 16)],
                        out_hbm.at[pl.ds(wrow, RPW), pl.ds(col, 16)])

    long_pass(idx_d, dis_t, 32, 0)
    long_pass(idx_p, phe_t, 16, 32)
    sub_pass(idx_sub, 48)


def _sc_featurize(idx_d, idx_p, idx_sub, dis_t, phe_t, sub_t):
    mesh = plsc.VectorSubcoreMesh(core_axis_name="c", subcore_axis_name="s")
    f = pl.kernel(
        _sc_featurize_body,
        out_type=jax.ShapeDtypeStruct((B, FEAT // 2), jnp.float32),
        mesh=mesh,
        compiler_params=pltpu.CompilerParams(use_tc_tiling_on_sc=False),
        scratch_types=[
            pltpu.VMEM_SHARED((13752, 16), jnp.int32),
            pltpu.VMEM_SHARED((17393, 16), jnp.float32),
            pltpu.VMEM_SHARED((30, 16), jnp.float32),
            pltpu.VMEM((SB, L), jnp.int32),
            pltpu.VMEM((RPW * LS,), jnp.int32),
            pltpu.VMEM((NB, 128, 16), jnp.int32),
            pltpu.VMEM((NB, 128, 16), jnp.float32),
            pltpu.VMEM((RPW, 32), jnp.float32),
        ] + [pltpu.SemaphoreType.DMA] * NB,
    )
    return f(idx_d, idx_p, idx_sub, dis_t, phe_t, sub_t)


def _mlp_body(x1_ref, x2_ref, w1a_ref, w1b_ref, b1_ref, w2_ref, b2_ref, w3_ref,
              b3_ref, o_ref):
    h = jnp.dot(x1_ref[...], w1a_ref[...], preferred_element_type=jnp.float32)
    h = h + jnp.dot(x2_ref[...], w1b_ref[...], preferred_element_type=jnp.float32)
    h = h + b1_ref[...]
    h = jnp.where(h >= 0, h, 0.01 * h)
    h = jnp.dot(h, w2_ref[...], preferred_element_type=jnp.float32)
    h = h + b2_ref[...]
    h = jnp.where(h >= 0, h, 0.01 * h)
    o = jnp.dot(h, w3_ref[...], preferred_element_type=jnp.float32)
    o_ref[...] = o + b3_ref[...]


def _mlp(x1, x2, W1, b1, W2, b2, W3, b3):
    BB = 1024
    HF = FEAT // 2
    return pl.pallas_call(
        _mlp_body,
        grid=(B // BB,),
        in_specs=[
            pl.BlockSpec((BB, HF), lambda i: (i, 0)),
            pl.BlockSpec((BB, HF), lambda i: (i, 0)),
            pl.BlockSpec((HF, H1), lambda i: (0, 0)),
            pl.BlockSpec((HF, H1), lambda i: (0, 0)),
            pl.BlockSpec((1, H1), lambda i: (0, 0)),
            pl.BlockSpec((H1, H2), lambda i: (0, 0)),
            pl.BlockSpec((1, H2), lambda i: (0, 0)),
            pl.BlockSpec((H2, 1), lambda i: (0, 0)),
            pl.BlockSpec((1, 1), lambda i: (0, 0)),
        ],
        out_specs=pl.BlockSpec((BB, 1), lambda i: (i, 0)),
        out_shape=jax.ShapeDtypeStruct((B, 1), jnp.float32),
    )(x1, x2, W1[:HF], W1[HF:], b1.reshape(1, H1), W2, b2.reshape(1, H2),
      W3, b3.reshape(1, 1))


def kernel(compound_diseases, compound_phenotypes, compound_subcellular_locations,
           protein_diseases, protein_phenotypes, protein_subcellular_locations,
           disease_table, phenotype_table, sub_table, W1, b1, W2, b2, W3, b3):
    cd = compound_diseases.astype(jnp.int32)
    cp = compound_phenotypes.astype(jnp.int32)
    cs = compound_subcellular_locations.reshape(-1).astype(jnp.int32)
    pd = protein_diseases.astype(jnp.int32)
    pp = protein_phenotypes.astype(jnp.int32)
    ps = protein_subcellular_locations.reshape(-1).astype(jnp.int32)
    perm = jnp.arange(32).reshape(2, 16).T.reshape(-1)
    dis_packed = jax.lax.bitcast_convert_type(
        disease_table.astype(jnp.bfloat16)[:, perm].reshape(-1, 16, 2),
        jnp.int32)
    x1 = _sc_featurize(cd, cp, cs, dis_packed, phenotype_table, sub_table)
    x2 = _sc_featurize(pd, pp, ps, dis_packed, phenotype_table, sub_table)
    return _mlp(x1, x2, W1, b1, W2, b2, W3, b3)


# speedup vs baseline: 1.1642x; 1.0345x over previous
import jax
import jax.numpy as jnp
from jax import lax
from jax.experimental import pallas as pl
from jax.experimental.pallas import tpu as pltpu
from jax.experimental.pallas import tpu_sc as plsc

B = 16384
L = 200
LS = 20
DD, DP, DS = 32, 16, 16
FEAT = (DD + DP + DS) * 2
H1, H2 = 128, 64

NC, NS = 2, 16
NW = NC * NS
RPW = B // NW
SB = 64
NSB = RPW // SB
NB = 8
K = 7


def _sc_featurize_body(idx_d, idx_p, dis_hbm, phe_hbm, out_hbm,
                       dis_t, phe_t, idx_v, rows32, rows16, stage, *sems):
    wid = lax.axis_index("s") * NC + lax.axis_index("c")
    wrow = wid * RPW
    sid = lax.axis_index("s")

    nd = dis_hbm.shape[0] // NS
    np_ = phe_hbm.shape[0] // NS
    pltpu.sync_copy(dis_hbm.at[pl.ds(sid * nd, nd), :], dis_t.at[pl.ds(sid * nd, nd), :])
    pltpu.sync_copy(phe_hbm.at[pl.ds(sid * np_, np_), :], phe_t.at[pl.ds(sid * np_, np_), :])

    @pl.when(sid == 0)
    def _():
        rem_d = dis_hbm.shape[0] - nd * NS
        rem_p = phe_hbm.shape[0] - np_ * NS
        pltpu.sync_copy(dis_hbm.at[pl.ds(nd * NS, rem_d), :], dis_t.at[pl.ds(nd * NS, rem_d), :])
        pltpu.sync_copy(phe_hbm.at[pl.ds(np_ * NS, rem_p), :], phe_t.at[pl.ds(np_ * NS, rem_p), :])

    plsc.subcore_barrier()

    def long_pass(idx_hbm, tab_sp, d, col):
        rows = rows32 if d == 32 else rows16
        inv = 1.0 / L
        NCH = SB * 2

        def fire(ch_q, ch_i, buf, sem):
            r = ch_q * 4 + ch_i // 2
            if ch_i % 2 == 0:
                pltpu.async_copy(tab_sp.at[idx_v.at[r, pl.ds(0, 128)]],
                                 rows.at[buf, pl.ds(0, 128), :], sem)
            else:
                pltpu.async_copy(tab_sp.at[idx_v.at[r, pl.ds(128, 72)]],
                                 rows.at[buf, pl.ds(0, 72), :], sem)

        def drain(part, buf, sem):
            n = 128 if part == 0 else 72
            pltpu.make_async_copy(tab_sp.at[idx_v.at[0, pl.ds(0, n)]],
                                  rows.at[buf, pl.ds(0, n), :], sem).wait()

        def reduce_chunk(buf, nrows, acc):
            if d == 32:
                def red(j, a):
                    a0, a1, b0, b1 = a
                    base = j * 8
                    for t in range(0, 8, 2):
                        x0 = rows[buf, base + t, :]
                        x1 = rows[buf, base + t + 1, :]
                        a0 = a0 + lax.bitcast_convert_type(x0 << 16, jnp.float32)
                        a1 = a1 + lax.bitcast_convert_type(x0 & jnp.int32(-65536), jnp.float32)
                        b0 = b0 + lax.bitcast_convert_type(x1 << 16, jnp.float32)
                        b1 = b1 + lax.bitcast_convert_type(x1 & jnp.int32(-65536), jnp.float32)
                    return a0, a1, b0, b1
            else:
                def red(j, a):
                    a0, b0 = a
                    base = j * 8
                    for t in range(0, 8, 2):
                        r0 = base + t
                        a0 = a0 + rows[buf, r0, pl.ds(0, 16)]
                        b0 = b0 + rows[buf, r0 + 1, pl.ds(0, 16)]
                    return a0, b0
            return lax.fori_loop(0, nrows // 8, red, acc)

        def store_row(row, acc):
            if d == 32:
                a0, a1, b0, b1 = acc
                stage[row, pl.ds(0, 16)] = (a0 + b0) * inv
                stage[row, pl.ds(16, 16)] = (a1 + b1) * inv
            else:
                a0, b0 = acc
                stage[row, pl.ds(0, 16)] = (a0 + b0) * inv

        zacc = (jnp.zeros((16,), jnp.float32),) * (4 if d == 32 else 2)

        def sblock_body(sb, _):
            row0 = wrow + sb * SB
            pltpu.sync_copy(idx_hbm.at[pl.ds(row0, SB), :], idx_v)
            for p in range(K):
                fire(0, p, p % NB, sems[p % NB])

            def oct_body(q, _):
                acc = zacc
                for i in range(NB):
                    fi = i + K
                    fbuf = (i + K) % NB

                    @pl.when(q * NB + fi < NCH)
                    def _():
                        fire(q, fi, fbuf, sems[fbuf])

                    drain(i % 2, i, sems[i])
                    acc = reduce_chunk(i, 128 if i % 2 == 0 else 72, acc)
                    if i % 2 == 1:
                        store_row(sb * SB + q * 4 + i // 2, acc)
                        acc = zacc
                return 0

            lax.fori_loop(0, NCH // NB, oct_body, 0)
            return 0

        lax.fori_loop(0, NSB, sblock_body, 0)
        src = stage if d == 32 else stage.at[:, pl.ds(0, 16)]
        pltpu.sync_copy(src, out_hbm.at[pl.ds(wrow, RPW), pl.ds(col, d)])

    long_pass(idx_d, dis_t, 32, 0)
    long_pass(idx_p, phe_t, 16, 32)


def _sc_sub_body(cs, ps, sub_hbm, out_hbm, sub_t, idx_s, rows16, stage, *sems):
    wid = lax.axis_index("s") * NC + lax.axis_index("c")
    wrow = wid * RPW
    sid = lax.axis_index("s")

    @pl.when(sid == 0)
    def _():
        pltpu.sync_copy(sub_hbm, sub_t)

    plsc.subcore_barrier()

    def fire_s(buf, off, sem):
        o = pl.multiple_of(off, 8)
        pltpu.async_copy(sub_t.at[idx_s.at[pl.ds(o, 80)]],
                         rows16.at[buf, pl.ds(0, 80), :], sem)

    def drain_s(buf, sem):
        pltpu.make_async_copy(sub_t.at[idx_s.at[pl.ds(0, 80)]],
                              rows16.at[buf, pl.ds(0, 80), :], sem).wait()

    def sub_pass(idx_hbm, col):
        wbase = wid * (RPW * LS)
        inv = 1.0 / LS
        nchunks = RPW // 4
        pltpu.sync_copy(idx_hbm.at[pl.ds(pl.multiple_of(wbase, 8), RPW * LS)],
                        idx_s)
        for p in range(4):
            fire_s(p % NB, p * 80, sems[p % NB])

        def oct_body(q, _):
            for i in range(NB):
                c = q * NB + i
                fc = c + 4
                fbuf = (i + 4) % NB

                @pl.when(fc < nchunks)
                def _():
                    fire_s(fbuf, fc * 80, sems[fbuf])

                drain_s(i, sems[i])
                for seg in range(4):
                    acc = jnp.zeros((16,), jnp.float32)
                    for j in range(LS):
                        acc = acc + rows16[i, seg * LS + j, pl.ds(0, 16)]
                    stage[c * 4 + seg, pl.ds(0, 16)] = acc * inv
            return 0

        lax.fori_loop(0, nchunks // NB, oct_body, 0)
        pltpu.sync_copy(stage, out_hbm.at[pl.ds(wrow, RPW), pl.ds(col, 16)])

    sub_pass(cs, 0)
    sub_pass(ps, 16)


def _sc_featurize(idx_d, idx_p, dis_t, phe_t):
    mesh = plsc.VectorSubcoreMesh(core_axis_name="c", subcore_axis_name="s")
    f = pl.kernel(
        _sc_featurize_body,
        out_type=jax.ShapeDtypeStruct((B, 48), jnp.float32),
        mesh=mesh,
        compiler_params=pltpu.CompilerParams(use_tc_tiling_on_sc=False),
        scratch_types=[
            pltpu.VMEM_SHARED((13752, 16), jnp.int32),
            pltpu.VMEM_SHARED((17393, 16), jnp.float32),
            pltpu.VMEM((SB, L), jnp.int32),
            pltpu.VMEM((NB, 128, 16), jnp.int32),
            pltpu.VMEM((NB, 128, 16), jnp.float32),
            pltpu.VMEM((RPW, 32), jnp.float32),
        ] + [pltpu.SemaphoreType.DMA] * NB,
    )
    return f(idx_d, idx_p, dis_t, phe_t)


def _sc_sub(cs, ps, sub_t):
    mesh = plsc.VectorSubcoreMesh(core_axis_name="c", subcore_axis_name="s")
    f = pl.kernel(
        _sc_sub_body,
        out_type=jax.ShapeDtypeStruct((B, 32), jnp.float32),
        mesh=mesh,
        compiler_params=pltpu.CompilerParams(use_tc_tiling_on_sc=False),
        scratch_types=[
            pltpu.VMEM_SHARED((30, 16), jnp.float32),
            pltpu.VMEM((RPW * LS,), jnp.int32),
            pltpu.VMEM((NB, 80, 16), jnp.float32),
            pltpu.VMEM((RPW, 16), jnp.float32),
        ] + [pltpu.SemaphoreType.DMA] * NB,
    )
    return f(cs, ps, sub_t)


def _mlp_body(x1_ref, x2_ref, x3_ref, w1a_ref, w1b_ref, w1c_ref, b1_ref, w2_ref,
              b2_ref, w3_ref, b3_ref, o_ref):
    h = jnp.dot(x1_ref[...], w1a_ref[...], preferred_element_type=jnp.float32)
    h = h + jnp.dot(x2_ref[...], w1b_ref[...], preferred_element_type=jnp.float32)
    h = h + jnp.dot(x3_ref[...], w1c_ref[...], preferred_element_type=jnp.float32)
    h = h + b1_ref[...]
    h = jnp.where(h >= 0, h, 0.01 * h)
    h = jnp.dot(h, w2_ref[...], preferred_element_type=jnp.float32)
    h = h + b2_ref[...]
    h = jnp.where(h >= 0, h, 0.01 * h)
    o = jnp.dot(h, w3_ref[...], preferred_element_type=jnp.float32)
    o_ref[...] = o + b3_ref[...]


def _mlp(x1, x2, x3, W1, b1, W2, b2, W3, b3):
    BB = 1024
    w1c = jnp.concatenate([W1[48:64], W1[112:128]], axis=0)
    return pl.pallas_call(
        _mlp_body,
        grid=(B // BB,),
        in_specs=[
            pl.BlockSpec((BB, 48), lambda i: (i, 0)),
            pl.BlockSpec((BB, 48), lambda i: (i, 0)),
            pl.BlockSpec((BB, 32), lambda i: (i, 0)),
            pl.BlockSpec((48, H1), lambda i: (0, 0)),
            pl.BlockSpec((48, H1), lambda i: (0, 0)),
            pl.BlockSpec((32, H1), lambda i: (0, 0)),
            pl.BlockSpec((1, H1), lambda i: (0, 0)),
            pl.BlockSpec((H1, H2), lambda i: (0, 0)),
            pl.BlockSpec((1, H2), lambda i: (0, 0)),
            pl.BlockSpec((H2, 1), lambda i: (0, 0)),
            pl.BlockSpec((1, 1), lambda i: (0, 0)),
        ],
        out_specs=pl.BlockSpec((BB, 1), lambda i: (i, 0)),
        out_shape=jax.ShapeDtypeStruct((B, 1), jnp.float32),
    )(x1, x2, x3, W1[0:48], W1[64:112], w1c, b1.reshape(1, H1), W2,
      b2.reshape(1, H2), W3, b3.reshape(1, 1))


def kernel(compound_diseases, compound_phenotypes, compound_subcellular_locations,
           protein_diseases, protein_phenotypes, protein_subcellular_locations,
           disease_table, phenotype_table, sub_table, W1, b1, W2, b2, W3, b3):
    cd = compound_diseases.astype(jnp.int32)
    cp = compound_phenotypes.astype(jnp.int32)
    cs = compound_subcellular_locations.reshape(-1).astype(jnp.int32)
    pd = protein_diseases.astype(jnp.int32)
    pp = protein_phenotypes.astype(jnp.int32)
    ps = protein_subcellular_locations.reshape(-1).astype(jnp.int32)
    perm = jnp.arange(32).reshape(2, 16).T.reshape(-1)
    dis_packed = jax.lax.bitcast_convert_type(
        disease_table.astype(jnp.bfloat16)[:, perm].reshape(-1, 16, 2),
        jnp.int32)
    x3 = _sc_sub(cs, ps, sub_table)
    x1 = _sc_featurize(cd, cp, dis_packed, phenotype_table)
    x2 = _sc_featurize(pd, pp, dis_packed, phenotype_table)
    return _mlp(x1, x2, x3, W1, b1, W2, b2, W3, b3)


# speedup vs baseline: 1.1739x; 1.0084x over previous
import jax
import jax.numpy as jnp
from jax import lax
from jax.experimental import pallas as pl
from jax.experimental.pallas import tpu as pltpu
from jax.experimental.pallas import tpu_sc as plsc

B = 16384
L = 200
LS = 20
DD, DP, DS = 32, 16, 16
FEAT = (DD + DP + DS) * 2
H1, H2 = 128, 64

NC, NS = 2, 16
NW = NC * NS
RPW = B // NW
SB = 64
NSB = RPW // SB
NB = 8
K = 7


def _make_long_body(d):
  def body(idx_hbm, tab_hbm, out_hbm, tab_sp, idx_v, rows, stage, *sems):
    wid = lax.axis_index("s") * NC + lax.axis_index("c")
    wrow = wid * RPW
    sid = lax.axis_index("s")

    nt = tab_hbm.shape[0] // NS
    pltpu.sync_copy(tab_hbm.at[pl.ds(sid * nt, nt), :], tab_sp.at[pl.ds(sid * nt, nt), :])

    @pl.when(sid == 0)
    def _():
        rem = tab_hbm.shape[0] - nt * NS
        pltpu.sync_copy(tab_hbm.at[pl.ds(nt * NS, rem), :], tab_sp.at[pl.ds(nt * NS, rem), :])

    plsc.subcore_barrier()

    def long_pass():
        inv = 1.0 / L
        NCH = SB * 2

        def fire(ch_q, ch_i, buf, sem):
            r = ch_q * 4 + ch_i // 2
            if ch_i % 2 == 0:
                pltpu.async_copy(tab_sp.at[idx_v.at[r, pl.ds(0, 128)]],
                                 rows.at[buf, pl.ds(0, 128), :], sem)
            else:
                pltpu.async_copy(tab_sp.at[idx_v.at[r, pl.ds(128, 72)]],
                                 rows.at[buf, pl.ds(0, 72), :], sem)

        def drain(part, buf, sem):
            n = 128 if part == 0 else 72
            pltpu.make_async_copy(tab_sp.at[idx_v.at[0, pl.ds(0, n)]],
                                  rows.at[buf, pl.ds(0, n), :], sem).wait()

        def reduce_chunk(buf, nrows, acc):
            if d == 32:
                def red(j, a):
                    a0, a1, b0, b1 = a
                    base = j * 8
                    for t in range(0, 8, 2):
                        x0 = rows[buf, base + t, :]
                        x1 = rows[buf, base + t + 1, :]
                        a0 = a0 + lax.bitcast_convert_type(x0 << 16, jnp.float32)
                        a1 = a1 + lax.bitcast_convert_type(x0 & jnp.int32(-65536), jnp.float32)
                        b0 = b0 + lax.bitcast_convert_type(x1 << 16, jnp.float32)
                        b1 = b1 + lax.bitcast_convert_type(x1 & jnp.int32(-65536), jnp.float32)
                    return a0, a1, b0, b1
            else:
                def red(j, a):
                    a0, b0 = a
                    base = j * 8
                    for t in range(0, 8, 2):
                        r0 = base + t
                        a0 = a0 + rows[buf, r0, pl.ds(0, 16)]
                        b0 = b0 + rows[buf, r0 + 1, pl.ds(0, 16)]
                    return a0, b0
            return lax.fori_loop(0, nrows // 8, red, acc)

        def store_row(row, acc):
            if d == 32:
                a0, a1, b0, b1 = acc
                stage[row, pl.ds(0, 16)] = (a0 + b0) * inv
                stage[row, pl.ds(16, 16)] = (a1 + b1) * inv
            else:
                a0, b0 = acc
                stage[row, pl.ds(0, 16)] = (a0 + b0) * inv

        zacc = (jnp.zeros((16,), jnp.float32),) * (4 if d == 32 else 2)

        def sblock_body(sb, _):
            row0 = wrow + sb * SB
            pltpu.sync_copy(idx_hbm.at[pl.ds(row0, SB), :], idx_v)
            for p in range(K):
                fire(0, p, p % NB, sems[p % NB])

            def oct_body(q, _):
                acc = zacc
                for i in range(NB):
                    fi = i + K
                    fbuf = (i + K) % NB

                    @pl.when(q * NB + fi < NCH)
                    def _():
                        fire(q, fi, fbuf, sems[fbuf])

                    drain(i % 2, i, sems[i])
                    acc = reduce_chunk(i, 128 if i % 2 == 0 else 72, acc)
                    if i % 2 == 1:
                        store_row(sb * SB + q * 4 + i // 2, acc)
                        acc = zacc
                return 0

            lax.fori_loop(0, NCH // NB, oct_body, 0)
            return 0

        lax.fori_loop(0, NSB, sblock_body, 0)
        pltpu.sync_copy(stage, out_hbm.at[pl.ds(wrow, RPW), :])

    long_pass()
  return body


def _sc_sub_body(cs, ps, sub_hbm, out_hbm, sub_t, idx_s, rows16, stage, *sems):
    wid = lax.axis_index("s") * NC + lax.axis_index("c")
    wrow = wid * RPW
    sid = lax.axis_index("s")

    @pl.when(sid == 0)
    def _():
        pltpu.sync_copy(sub_hbm, sub_t)

    plsc.subcore_barrier()

    def fire_s(buf, off, sem):
        o = pl.multiple_of(off, 8)
        pltpu.async_copy(sub_t.at[idx_s.at[pl.ds(o, 80)]],
                         rows16.at[buf, pl.ds(0, 80), :], sem)

    def drain_s(buf, sem):
        pltpu.make_async_copy(sub_t.at[idx_s.at[pl.ds(0, 80)]],
                              rows16.at[buf, pl.ds(0, 80), :], sem).wait()

    def sub_pass(idx_hbm, col):
        wbase = wid * (RPW * LS)
        inv = 1.0 / LS
        nchunks = RPW // 4
        pltpu.sync_copy(idx_hbm.at[pl.ds(pl.multiple_of(wbase, 8), RPW * LS)],
                        idx_s)
        for p in range(4):
            fire_s(p % NB, p * 80, sems[p % NB])

        def oct_body(q, _):
            for i in range(NB):
                c = q * NB + i
                fc = c + 4
                fbuf = (i + 4) % NB

                @pl.when(fc < nchunks)
                def _():
                    fire_s(fbuf, fc * 80, sems[fbuf])

                drain_s(i, sems[i])
                for seg in range(4):
                    acc = jnp.zeros((16,), jnp.float32)
                    for j in range(LS):
                        acc = acc + rows16[i, seg * LS + j, pl.ds(0, 16)]
                    stage[c * 4 + seg, pl.ds(0, 16)] = acc * inv
            return 0

        lax.fori_loop(0, nchunks // NB, oct_body, 0)
        pltpu.sync_copy(stage, out_hbm.at[pl.ds(wrow, RPW), pl.ds(col, 16)])

    sub_pass(cs, 0)
    sub_pass(ps, 16)


_dis_body = _make_long_body(32)
_phe_body = _make_long_body(16)


def _sc_long(idx, tab, d):
    mesh = plsc.VectorSubcoreMesh(core_axis_name="c", subcore_axis_name="s")
    if d == 32:
        body = _dis_body
        tab_t = pltpu.VMEM_SHARED((13752, 16), jnp.int32)
        rows_t = pltpu.VMEM((NB, 128, 16), jnp.int32)
    else:
        body = _phe_body
        tab_t = pltpu.VMEM_SHARED((17393, 16), jnp.float32)
        rows_t = pltpu.VMEM((NB, 128, 16), jnp.float32)
    f = pl.kernel(
        body,
        out_type=jax.ShapeDtypeStruct((B, d), jnp.float32),
        mesh=mesh,
        compiler_params=pltpu.CompilerParams(use_tc_tiling_on_sc=False),
        scratch_types=[
            tab_t,
            pltpu.VMEM((SB, L), jnp.int32),
            rows_t,
            pltpu.VMEM((RPW, d), jnp.float32),
        ] + [pltpu.SemaphoreType.DMA] * NB,
    )
    return f(idx, tab)


def _sc_sub(cs, ps, sub_t):
    mesh = plsc.VectorSubcoreMesh(core_axis_name="c", subcore_axis_name="s")
    f = pl.kernel(
        _sc_sub_body,
        out_type=jax.ShapeDtypeStruct((B, 32), jnp.float32),
        mesh=mesh,
        compiler_params=pltpu.CompilerParams(use_tc_tiling_on_sc=False),
        scratch_types=[
            pltpu.VMEM_SHARED((30, 16), jnp.float32),
            pltpu.VMEM((RPW * LS,), jnp.int32),
            pltpu.VMEM((NB, 80, 16), jnp.float32),
            pltpu.VMEM((RPW, 16), jnp.float32),
        ] + [pltpu.SemaphoreType.DMA] * NB,
    )
    return f(cs, ps, sub_t)


def _mlp_body(x1_ref, x2_ref, x3_ref, x4_ref, x5_ref, w1a_ref, w1b_ref, w1c_ref,
              w1d_ref, w1e_ref, b1_ref, w2_ref, b2_ref, w3_ref, b3_ref, o_ref):
    h = jnp.dot(x1_ref[...], w1a_ref[...], preferred_element_type=jnp.float32)
    h = h + jnp.dot(x2_ref[...], w1b_ref[...], preferred_element_type=jnp.float32)
    h = h + jnp.dot(x3_ref[...], w1c_ref[...], preferred_element_type=jnp.float32)
    h = h + jnp.dot(x4_ref[...], w1d_ref[...], preferred_element_type=jnp.float32)
    h = h + jnp.dot(x5_ref[...], w1e_ref[...], preferred_element_type=jnp.float32)
    h = h + b1_ref[...]
    h = jnp.where(h >= 0, h, 0.01 * h)
    h = jnp.dot(h, w2_ref[...], preferred_element_type=jnp.float32)
    h = h + b2_ref[...]
    h = jnp.where(h >= 0, h, 0.01 * h)
    o = jnp.dot(h, w3_ref[...], preferred_element_type=jnp.float32)
    o_ref[...] = o + b3_ref[...]


def _mlp(x1, x2, x3, x4, x5, W1, b1, W2, b2, W3, b3):
    BB = 1024
    w1e = jnp.concatenate([W1[48:64], W1[112:128]], axis=0)
    return pl.pallas_call(
        _mlp_body,
        grid=(B // BB,),
        in_specs=[
            pl.BlockSpec((BB, 32), lambda i: (i, 0)),
            pl.BlockSpec((BB, 16), lambda i: (i, 0)),
            pl.BlockSpec((BB, 32), lambda i: (i, 0)),
            pl.BlockSpec((BB, 16), lambda i: (i, 0)),
            pl.BlockSpec((BB, 32), lambda i: (i, 0)),
            pl.BlockSpec((32, H1), lambda i: (0, 0)),
            pl.BlockSpec((16, H1), lambda i: (0, 0)),
            pl.BlockSpec((32, H1), lambda i: (0, 0)),
            pl.BlockSpec((16, H1), lambda i: (0, 0)),
            pl.BlockSpec((32, H1), lambda i: (0, 0)),
            pl.BlockSpec((1, H1), lambda i: (0, 0)),
            pl.BlockSpec((H1, H2), lambda i: (0, 0)),
            pl.BlockSpec((1, H2), lambda i: (0, 0)),
            pl.BlockSpec((H2, 1), lambda i: (0, 0)),
            pl.BlockSpec((1, 1), lambda i: (0, 0)),
        ],
        out_specs=pl.BlockSpec((BB, 1), lambda i: (i, 0)),
        out_shape=jax.ShapeDtypeStruct((B, 1), jnp.float32),
    )(x1, x2, x3, x4, x5, W1[0:32], W1[32:48], W1[64:96], W1[96:112], w1e,
      b1.reshape(1, H1), W2, b2.reshape(1, H2), W3, b3.reshape(1, 1))


def kernel(compound_diseases, compound_phenotypes, compound_subcellular_locations,
           protein_diseases, protein_phenotypes, protein_subcellular_locations,
           disease_table, phenotype_table, sub_table, W1, b1, W2, b2, W3, b3):
    cd = compound_diseases.astype(jnp.int32)
    cp = compound_phenotypes.astype(jnp.int32)
    cs = compound_subcellular_locations.reshape(-1).astype(jnp.int32)
    pd = protein_diseases.astype(jnp.int32)
    pp = protein_phenotypes.astype(jnp.int32)
    ps = protein_subcellular_locations.reshape(-1).astype(jnp.int32)
    perm = jnp.arange(32).reshape(2, 16).T.reshape(-1)
    dis_packed = jax.lax.bitcast_convert_type(
        disease_table.astype(jnp.bfloat16)[:, perm].reshape(-1, 16, 2),
        jnp.int32)
    x5 = _sc_sub(cs, ps, sub_table)
    x1 = _sc_long(cd, dis_packed, 32)
    x2 = _sc_long(cp, phenotype_table, 16)
    x3 = _sc_long(pd, dis_packed, 32)
    x4 = _sc_long(pp, phenotype_table, 16)
    return _mlp(x1, x2, x3, x4, x5, W1, b1, W2, b2, W3, b3)
